# baseline (speedup 1.0000x reference)
import jax
import jax.numpy as jnp
from jax import lax
from jax.experimental import pallas as pl
from jax.experimental.pallas import tpu as pltpu
from jax.experimental.pallas import tpu_sc as plsc

D = 128
B = 16384
NC = 2
NS = 16
L = 16
NW = NC * NS
BPW = B // NW
CB = 128
NCHUNK = BPW // CB


def _body(idx0_hbm, idx1_hbm, w_hbm, out_hbm,
          idx0_v, idx1_v, rows0, rows1, out_v, s0, s1, s2, s3):
    wid = lax.axis_index("s") * NC + lax.axis_index("c")
    base = wid * BPW
    sems = (s0, s1, s2, s3)

    pltpu.sync_copy(idx0_hbm.at[pl.ds(base, BPW)], idx0_v)
    pltpu.sync_copy(idx1_hbm.at[pl.ds(base, BPW)], idx1_v)
    for c in range(NCHUNK):
        sl = pl.ds(c * CB, CB)
        pltpu.async_copy(w_hbm.at[idx0_v.at[sl]], rows0.at[sl], sems[c])
        pltpu.async_copy(w_hbm.at[idx1_v.at[sl]], rows1.at[sl], sems[c])

    lanes = lax.iota(jnp.int32, L)
    last_lane = lanes == (L - 1)

    for c in range(NCHUNK):
        sl = pl.ds(c * CB, CB)
        pltpu.make_async_copy(w_hbm.at[idx0_v.at[sl]], rows0.at[sl],
                              sems[c]).wait()
        pltpu.make_async_copy(w_hbm.at[idx1_v.at[sl]], rows1.at[sl],
                              sems[c]).wait()

        @plsc.parallel_loop(0, CB, 1, unroll=2)
        def _(e, c=c):
            eg = c * CB + e
            acc0 = jnp.zeros((L,), jnp.float32)
            acc1 = jnp.zeros((L,), jnp.float32)
            for s in range(D // (2 * L)):
                x0 = rows0[eg, pl.ds(s * 2 * L, 2 * L)]
                x1 = rows1[eg, pl.ds(s * 2 * L, 2 * L)]
                p = x0 * x1
                a, b = plsc.unpack(p, format=plsc.PackFormat.INTERLEAVED)
                acc0 = acc0 + a
                acc1 = acc1 + b
            scn = plsc.cumsum(acc0 + acc1)
            pos = jnp.full((L,), eg, jnp.int32)
            plsc.store_scatter(out_v, [pos], scn, mask=last_lane)

    pltpu.sync_copy(out_v, out_hbm.at[pl.ds(base, BPW)])


def kernel(X_idxs, W):
    idx0 = X_idxs[:, 0].astype(jnp.int32)
    idx1 = X_idxs[:, 1].astype(jnp.int32)
    w_bf = W.astype(jnp.bfloat16)
    mesh = plsc.VectorSubcoreMesh(core_axis_name="c", subcore_axis_name="s")
    f = pl.kernel(
        _body,
        out_type=jax.ShapeDtypeStruct((B,), jnp.float32),
        mesh=mesh,
        compiler_params=pltpu.CompilerParams(
            needs_layout_passes=False, use_tc_tiling_on_sc=False,
            disable_bounds_checks=True),
        scratch_types=[
            pltpu.VMEM((BPW,), jnp.int32),
            pltpu.VMEM((BPW,), jnp.int32),
            pltpu.VMEM((BPW, D), jnp.bfloat16),
            pltpu.VMEM((BPW, D), jnp.bfloat16),
            pltpu.VMEM((BPW,), jnp.float32),
            pltpu.SemaphoreType.DMA,
            pltpu.SemaphoreType.DMA,
            pltpu.SemaphoreType.DMA,
            pltpu.SemaphoreType.DMA,
        ],
    )
    return f(idx0, idx1, w_bf)
